# baseline (speedup 1.0000x reference)
import functools
import math

import jax
import jax.numpy as jnp
from jax import lax
from jax.experimental import pallas as pl
from jax.experimental.pallas import tpu as pltpu
from jax.experimental.pallas import tpu_sc as plsc

NS = 48
SH = 9
LW = 128
_HOFF = 48

EDGE_BLK = 2000

_MM_PREC = lax.Precision.DEFAULT


def _dot(a, b):
    return jnp.dot(a, b, precision=_MM_PREC, preferred_element_type=jnp.float32)


def _node_embed_body(x_ref, sig_ref, pos_ref, wx_ref, ws_ref, b_ref,
                     o_ref, ot_ref):
    acc = _dot(x_ref[...], wx_ref[...]) + _dot(sig_ref[...], ws_ref[...])
    h = jnp.maximum(acc + b_ref[...], 0.0)
    o_ref[...] = h
    nn = h.shape[0]
    ot_ref[...] = jnp.concatenate(
        [pos_ref[...], jnp.zeros((nn, 5), jnp.float32), sig_ref[...],
         jnp.zeros((nn, _HOFF - 40), jnp.float32), h,
         jnp.zeros((nn, LW - _HOFF - NS), jnp.float32)], axis=1)


def _node_embed(x, sig, pos, W_node, b_node):
    n = x.shape[0]
    return pl.pallas_call(
        _node_embed_body,
        out_shape=[jax.ShapeDtypeStruct((n, NS), jnp.float32),
                   jax.ShapeDtypeStruct((n, LW), jnp.float32)],
    )(x, sig, pos, W_node[:, :16].T, W_node[:, 16:].T, b_node.reshape(1, NS))


def _edge_prep_body(ea_ref, ps_ref, pd_ref, w1a_ref, w1b_ref, w1c_ref,
                    b1_ref, w2_ref, b2_ref, e_ref, dv_ref):
    ps = ps_ref[...]
    pd = pd_ref[...]
    vec = pd[:, 0:3] - ps[:, 0:3]
    d2 = jnp.sum(vec * vec, axis=1, keepdims=True)
    d = jnp.sqrt(d2)
    offs = lax.broadcasted_iota(jnp.int32, (1, 32), 1).astype(jnp.float32) \
        * (5.0 / 31.0)
    coeff = -0.5 / (5.0 / 31.0) ** 2
    smear = jnp.exp(coeff * (d - offs) ** 2)
    acc = (_dot(ea_ref[...], w1a_ref[...])
           + _dot(ps[:, 8:40], w1b_ref[...])
           + _dot(smear, w1c_ref[...]))
    h1 = jnp.maximum(acc + b1_ref[...], 0.0)
    e_ref[...] = _dot(h1, w2_ref[...]) + b2_ref[...]
    dvec = vec / jnp.maximum(d, 1e-8)
    blk = dvec.shape[0]
    dx, dy, dz = dvec[:, 0:1], dvec[:, 1:2], dvec[:, 2:3]
    s3 = math.sqrt(3.0)
    s15 = math.sqrt(15.0)
    s5 = math.sqrt(5.0)
    dv_ref[...] = jnp.concatenate(
        [jnp.ones_like(dx), s3 * dx, s3 * dy, s3 * dz,
         s15 * dx * dy, s15 * dy * dz,
         (s5 / 2.0) * (3.0 * dz * dz - 1.0), s15 * dx * dz,
         (s15 / 2.0) * (dx * dx - dy * dy),
         jnp.zeros((blk, 7), jnp.float32)], axis=1)


def _edge_prep(edge_attr8, gsrc, gdst, We1, be1, We2, be2):
    e_count = edge_attr8.shape[0]
    grid = (e_count // EDGE_BLK,)
    eb = lambda w: pl.BlockSpec((EDGE_BLK, w), lambda i: (i, 0))
    wb = lambda s: pl.BlockSpec(s, lambda i: (0, 0))
    return pl.pallas_call(
        _edge_prep_body,
        grid=grid,
        in_specs=[eb(8), eb(LW), eb(LW),
                  wb((8, NS)), wb((32, NS)), wb((32, NS)), wb((1, NS)),
                  wb((NS, NS)), wb((1, NS))],
        out_specs=[eb(NS), eb(16)],
        out_shape=[jax.ShapeDtypeStruct((e_count, NS), jnp.float32),
                   jax.ShapeDtypeStruct((e_count, 16), jnp.float32)],
        compiler_params=pltpu.CompilerParams(
            dimension_semantics=("parallel",)),
    )(edge_attr8, gsrc, gdst,
      jnp.concatenate([We1[:, :5].T, jnp.zeros((3, NS), jnp.float32)], axis=0),
      We1[:, 5:37].T, We1[:, 37:69].T, be1.reshape(1, NS),
      We2.T, be2.reshape(1, NS))


def _conv_body(hoff, e_ref, hs_ref, hd_ref, sh_ref, w1_ref, b1_ref, w2_ref,
               b2_ref, p_ref, s_ref, o_ref):
    bf = jnp.bfloat16
    hs = hs_ref[:, hoff:hoff + NS].astype(jnp.float32)
    hd = hd_ref[:, hoff:hoff + NS].astype(jnp.float32)
    feat = jnp.concatenate([e_ref[...], hs, hd], axis=1)
    a = jnp.maximum(_dot(feat.astype(bf), w1_ref[...]) + b1_ref[...], 0.0)
    w2 = _dot(a.astype(bf), w2_ref[...]) + b2_ref[...]
    shexp = _dot(sh_ref[...].astype(bf), p_ref[...])
    t = _dot((w2 * shexp).astype(bf), s_ref[...])
    blk = t.shape[0]
    o_ref[...] = jnp.concatenate(
        [hs * t, jnp.ones((blk, 1), jnp.float32),
         jnp.zeros((blk, LW - NS - 1), jnp.float32)], axis=1)


def _conv_layer(e, hsrc, hdst, sh16, Wc1, bc1, Wc2, bc2, hoff):
    e_count = e.shape[0]
    grid = (e_count // EDGE_BLK,)
    eb = lambda w: pl.BlockSpec((EDGE_BLK, w), lambda i: (i, 0))
    wb = lambda s: pl.BlockSpec(s, lambda i: (0, 0))
    bf = jnp.bfloat16
    jj = jnp.arange(NS * SH) % SH
    pmat = (jnp.arange(16)[:, None] == jj[None, :]).astype(bf)
    ii = jnp.arange(NS * SH) // SH
    smat = (ii[:, None] == jnp.arange(NS)[None, :]).astype(bf)
    return pl.pallas_call(
        functools.partial(_conv_body, hoff),
        grid=grid,
        in_specs=[eb(NS), eb(LW), eb(LW), eb(16),
                  wb((3 * NS, 3 * NS)), wb((1, 3 * NS)),
                  wb((3 * NS, NS * SH)), wb((1, NS * SH)),
                  wb((16, NS * SH)), wb((NS * SH, NS))],
        out_specs=eb(LW),
        out_shape=jax.ShapeDtypeStruct((e_count, LW), jnp.float32),
        compiler_params=pltpu.CompilerParams(
            dimension_semantics=("parallel",)),
    )(e, hsrc, hdst, sh16, Wc1.T.astype(bf), bc1.reshape(1, -1),
      Wc2.T.astype(bf), bc2.reshape(1, -1), pmat, smat)


def _update_body(final, h_ref, acc_ref, *o_refs):
    nn = h_ref.shape[0]
    agg = acc_ref[0, 0:nn, 0:NS] + acc_ref[1, 0:nn, 0:NS]
    deg = jnp.maximum(
        acc_ref[0, 0:nn, NS:NS + 1] + acc_ref[1, 0:nn, NS:NS + 1], 1.0)
    hn = h_ref[...] + agg / deg
    o_refs[0][...] = hn
    if not final:
        o_refs[1][...] = jnp.concatenate(
            [hn, jnp.zeros((nn, LW - NS), jnp.float32)], axis=1)


def _h_update(h, acc, final):
    n = h.shape[0]
    shapes = [jax.ShapeDtypeStruct((n, NS), jnp.float32)]
    if not final:
        shapes.append(jax.ShapeDtypeStruct((n, LW), jnp.float32))
    out = pl.pallas_call(
        functools.partial(_update_body, final),
        out_shape=shapes,
    )(h, acc)
    return out if not final else (out[0], None)


_SC_CORES = 2
_SC_SUBCORES = 16
_SC_WORKERS = _SC_CORES * _SC_SUBCORES
_STREAM = 80
_CHUNKS = 1


def _sc_mesh():
    return plsc.VectorSubcoreMesh(core_axis_name="c", subcore_axis_name="s")


_N_PAD = 10240


@jax.jit
def _sc_gather2(tab_a, idx_a3, tab_b, idx_b3):
    _, rows_w, _ = idx_a3.shape
    e_count = _SC_WORKERS * rows_w * _STREAM
    dt_a, dt_b = tab_a.dtype, tab_b.dtype

    @functools.partial(
        pl.kernel,
        out_type=[jax.ShapeDtypeStruct((e_count, LW), dt_a),
                  jax.ShapeDtypeStruct((e_count, LW), dt_b)],
        mesh=_sc_mesh(),
        scratch_types=[pltpu.VMEM((rows_w, _STREAM), jnp.int32),
                       pltpu.VMEM((_STREAM, LW), dt_a),
                       pltpu.VMEM((_STREAM, LW), dt_a),
                       pltpu.VMEM((rows_w, _STREAM), jnp.int32),
                       pltpu.VMEM((_STREAM, LW), dt_b),
                       pltpu.VMEM((_STREAM, LW), dt_b),
                       pltpu.SemaphoreType.DMA,
                       pltpu.SemaphoreType.DMA,
                       pltpu.SemaphoreType.DMA,
                       pltpu.SemaphoreType.DMA],
    )
    def k(ta_hbm, ia_hbm, tb_hbm, ib_hbm, oa_hbm, ob_hbm,
          ia_v, ra0_v, ra1_v, ib_v, rb0_v, rb1_v, sa0, sa1, sb0, sb1):
        wid = lax.axis_index("s") * _SC_CORES + lax.axis_index("c")
        base = wid * rows_w * _STREAM
        pltpu.sync_copy(ia_hbm.at[wid], ia_v)
        pltpu.sync_copy(ib_hbm.at[wid], ib_v)

        def row(j, ra, rb, sa, sb):
            off = base + j * _STREAM
            ca = pltpu.async_copy(ta_hbm.at[ia_v.at[j]], ra, sa)
            cb = pltpu.async_copy(tb_hbm.at[ib_v.at[j]], rb, sb)
            return off, ca, cb

        @pl.loop(0, rows_w // 2)
        def _(p):
            j = p * 2
            o0, ca0, cb0 = row(j, ra0_v, rb0_v, sa0, sb0)
            o1, ca1, cb1 = row(j + 1, ra1_v, rb1_v, sa1, sb1)
            ca0.wait()
            pltpu.sync_copy(ra0_v, oa_hbm.at[pl.ds(o0, _STREAM)])
            cb0.wait()
            pltpu.sync_copy(rb0_v, ob_hbm.at[pl.ds(o0, _STREAM)])
            ca1.wait()
            pltpu.sync_copy(ra1_v, oa_hbm.at[pl.ds(o1, _STREAM)])
            cb1.wait()
            pltpu.sync_copy(rb1_v, ob_hbm.at[pl.ds(o1, _STREAM)])

        if rows_w % 2:
            o0, ca0, cb0 = row(rows_w - 1, ra0_v, rb0_v, sa0, sb0)
            ca0.wait()
            pltpu.sync_copy(ra0_v, oa_hbm.at[pl.ds(o0, _STREAM)])
            cb0.wait()
            pltpu.sync_copy(rb0_v, ob_hbm.at[pl.ds(o0, _STREAM)])

    return k(tab_a, idx_a3, tab_b, idx_b3)


@jax.jit
def _sc_scatter(msg, dst3, init):
    _, rows_w, _ = dst3.shape
    stripe = _N_PAD // _SC_SUBCORES

    @functools.partial(
        pl.kernel,
        out_type=jax.ShapeDtypeStruct((_SC_CORES, _N_PAD, LW), jnp.float32),
        mesh=_sc_mesh(),
        scratch_types=[pltpu.VMEM((rows_w, _STREAM), jnp.int32),
                       pltpu.VMEM((_STREAM, LW), jnp.float32),
                       pltpu.VMEM((_STREAM, LW), jnp.float32),
                       pltpu.VMEM_SHARED((_N_PAD, LW), jnp.float32),
                       pltpu.SemaphoreType.DMA,
                       pltpu.SemaphoreType.DMA],
    )
    def k(msg_hbm, dst_hbm, z_hbm, acc_hbm, idx_v, m0_v, m1_v, acc_sh,
          s0, s1):
        c = lax.axis_index("c")
        s = lax.axis_index("s")
        wid = s * _SC_CORES + c
        base = wid * rows_w * _STREAM
        pltpu.sync_copy(z_hbm.at[c, pl.ds(s * stripe, stripe)],
                        acc_sh.at[pl.ds(s * stripe, stripe)])
        pltpu.sync_copy(dst_hbm.at[wid], idx_v)
        plsc.subcore_barrier()

        @pl.loop(0, rows_w // 2)
        def _(p):
            j = p * 2
            c0 = pltpu.async_copy(
                msg_hbm.at[pl.ds(base + j * _STREAM, _STREAM)], m0_v, s0)
            c1 = pltpu.async_copy(
                msg_hbm.at[pl.ds(base + (j + 1) * _STREAM, _STREAM)], m1_v,
                s1)
            c0.wait()
            pltpu.sync_copy(m0_v, acc_sh.at[idx_v.at[j]], add=True)
            c1.wait()
            pltpu.sync_copy(m1_v, acc_sh.at[idx_v.at[j + 1]], add=True)

        if rows_w % 2:
            j = rows_w - 1
            pltpu.sync_copy(msg_hbm.at[pl.ds(base + j * _STREAM, _STREAM)],
                            m0_v)
            pltpu.sync_copy(m0_v, acc_sh.at[idx_v.at[j]], add=True)

        plsc.subcore_barrier()
        pltpu.sync_copy(acc_sh.at[pl.ds(s * stripe, stripe)],
                        acc_hbm.at[c, pl.ds(s * stripe, stripe)])

    return k(msg, dst3, init)


def kernel(x, pos, node_sigma_emb, edge_attr, W_node, b_node, We1, be1, We2,
           be2, Wc1_0, bc1_0, Wc2_0, bc2_0, Wc1_1, bc1_1, Wc2_1, bc2_1,
           edge_index):
    n = x.shape[0]
    e_count = edge_index.shape[1]
    e_chunk = e_count // _CHUNKS
    src3 = [edge_index[0, k * e_chunk:(k + 1) * e_chunk]
            .reshape(_SC_WORKERS, -1, _STREAM) for k in range(_CHUNKS)]
    dst3 = [edge_index[1, k * e_chunk:(k + 1) * e_chunk]
            .reshape(_SC_WORKERS, -1, _STREAM) for k in range(_CHUNKS)]

    h, hb = _node_embed(x, node_sigma_emb, pos, W_node, b_node)
    ea8 = jnp.concatenate(
        [edge_attr, jnp.zeros((e_count, 3), jnp.float32)], axis=1)

    zeros2 = jnp.zeros((_SC_CORES, _N_PAD, LW), jnp.float32)

    e, sh16 = [], []
    g = [_sc_gather2(hb, src3[k], hb, dst3[k]) for k in range(_CHUNKS)]
    acc = zeros2
    for k in range(_CHUNKS):
        tsrc, tdst = g[k]
        ek, shk = _edge_prep(ea8[k * e_chunk:(k + 1) * e_chunk], tsrc, tdst,
                             We1, be1, We2, be2)
        e.append(ek)
        sh16.append(shk)
        msg = _conv_layer(ek, tsrc, tdst, shk, Wc1_0, bc1_0, Wc2_0, bc2_0,
                          hoff=_HOFF)
        acc = _sc_scatter(msg, dst3[k], acc)
    h, hb = _h_update(h, acc, final=False)

    g = [_sc_gather2(hb, src3[k], hb, dst3[k]) for k in range(_CHUNKS)]
    acc = zeros2
    for k in range(_CHUNKS):
        hsrc, hdst = g[k]
        msg = _conv_layer(e[k], hsrc, hdst, sh16[k], Wc1_1, bc1_1,
                          Wc2_1, bc2_1, hoff=0)
        acc = _sc_scatter(msg, dst3[k], acc)
    h, _ = _h_update(h, acc, final=True)
    return h


# speedup vs baseline: 1.1785x; 1.1785x over previous
import functools
import math

import jax
import jax.numpy as jnp
from jax import lax
from jax.experimental import pallas as pl
from jax.experimental.pallas import tpu as pltpu
from jax.experimental.pallas import tpu_sc as plsc

NS = 48
SH = 9
LW = 128
_HOFF = 48

EDGE_BLK = 4000

_MM_PREC = lax.Precision.DEFAULT


def _dot(a, b):
    return jnp.dot(a, b, precision=_MM_PREC, preferred_element_type=jnp.float32)


def _node_embed_body(x_ref, sig_ref, pos_ref, wx_ref, ws_ref, b_ref,
                     o_ref, ot_ref):
    acc = _dot(x_ref[...], wx_ref[...]) + _dot(sig_ref[...], ws_ref[...])
    h = jnp.maximum(acc + b_ref[...], 0.0)
    o_ref[...] = h
    nn = h.shape[0]
    ot_ref[...] = jnp.concatenate(
        [pos_ref[...], jnp.zeros((nn, 5), jnp.float32), sig_ref[...],
         jnp.zeros((nn, _HOFF - 40), jnp.float32), h,
         jnp.zeros((nn, LW - _HOFF - NS), jnp.float32)], axis=1)


def _node_embed(x, sig, pos, W_node, b_node):
    n = x.shape[0]
    return pl.pallas_call(
        _node_embed_body,
        out_shape=[jax.ShapeDtypeStruct((n, NS), jnp.float32),
                   jax.ShapeDtypeStruct((n, LW), jnp.float32)],
    )(x, sig, pos, W_node[:, :16].T, W_node[:, 16:].T, b_node.reshape(1, NS))


def _edge_prep_body(ea_ref, ps_ref, pd_ref, w1a_ref, w1b_ref, w1c_ref,
                    b1_ref, w2_ref, b2_ref, e_ref, dv_ref):
    ps = ps_ref[...]
    pd = pd_ref[...]
    vec = pd[:, 0:3] - ps[:, 0:3]
    d2 = jnp.sum(vec * vec, axis=1, keepdims=True)
    d = jnp.sqrt(d2)
    offs = lax.broadcasted_iota(jnp.int32, (1, 32), 1).astype(jnp.float32) \
        * (5.0 / 31.0)
    coeff = -0.5 / (5.0 / 31.0) ** 2
    smear = jnp.exp(coeff * (d - offs) ** 2)
    acc = (_dot(ea_ref[...], w1a_ref[...])
           + _dot(ps[:, 8:40], w1b_ref[...])
           + _dot(smear, w1c_ref[...]))
    h1 = jnp.maximum(acc + b1_ref[...], 0.0)
    e_ref[...] = _dot(h1, w2_ref[...]) + b2_ref[...]
    dvec = vec / jnp.maximum(d, 1e-8)
    blk = dvec.shape[0]
    dx, dy, dz = dvec[:, 0:1], dvec[:, 1:2], dvec[:, 2:3]
    s3 = math.sqrt(3.0)
    s15 = math.sqrt(15.0)
    s5 = math.sqrt(5.0)
    dv_ref[...] = jnp.concatenate(
        [jnp.ones_like(dx), s3 * dx, s3 * dy, s3 * dz,
         s15 * dx * dy, s15 * dy * dz,
         (s5 / 2.0) * (3.0 * dz * dz - 1.0), s15 * dx * dz,
         (s15 / 2.0) * (dx * dx - dy * dy),
         jnp.zeros((blk, 7), jnp.float32)], axis=1)


def _edge_prep(edge_attr8, gsrc, gdst, We1, be1, We2, be2):
    e_count = edge_attr8.shape[0]
    grid = (e_count // EDGE_BLK,)
    eb = lambda w: pl.BlockSpec((EDGE_BLK, w), lambda i: (i, 0))
    wb = lambda s: pl.BlockSpec(s, lambda i: (0, 0))
    return pl.pallas_call(
        _edge_prep_body,
        grid=grid,
        in_specs=[eb(8), eb(LW), eb(LW),
                  wb((8, NS)), wb((32, NS)), wb((32, NS)), wb((1, NS)),
                  wb((NS, NS)), wb((1, NS))],
        out_specs=[eb(NS), eb(16)],
        out_shape=[jax.ShapeDtypeStruct((e_count, NS), jnp.float32),
                   jax.ShapeDtypeStruct((e_count, 16), jnp.float32)],
        compiler_params=pltpu.CompilerParams(
            dimension_semantics=("parallel",)),
    )(edge_attr8, gsrc, gdst,
      jnp.concatenate([We1[:, :5].T, jnp.zeros((3, NS), jnp.float32)], axis=0),
      We1[:, 5:37].T, We1[:, 37:69].T, be1.reshape(1, NS),
      We2.T, be2.reshape(1, NS))


def _conv_body(hoff, e_ref, hs_ref, hd_ref, sh_ref, w1_ref, b1_ref, w2_ref,
               b2_ref, p_ref, s_ref, o_ref):
    bf = jnp.bfloat16
    hs = hs_ref[:, hoff:hoff + NS].astype(jnp.float32)
    hd = hd_ref[:, hoff:hoff + NS].astype(jnp.float32)
    feat = jnp.concatenate([e_ref[...], hs, hd], axis=1)
    a = jnp.maximum(_dot(feat.astype(bf), w1_ref[...]) + b1_ref[...], 0.0)
    w2 = _dot(a.astype(bf), w2_ref[...]) + b2_ref[...]
    shexp = _dot(sh_ref[...].astype(bf), p_ref[...])
    t = _dot((w2 * shexp).astype(bf), s_ref[...])
    blk = t.shape[0]
    o_ref[...] = jnp.concatenate(
        [hs * t, jnp.ones((blk, 1), jnp.float32),
         jnp.zeros((blk, LW - NS - 1), jnp.float32)], axis=1)


def _conv_layer(e, hsrc, hdst, sh16, Wc1, bc1, Wc2, bc2, hoff):
    e_count = e.shape[0]
    grid = (e_count // EDGE_BLK,)
    eb = lambda w: pl.BlockSpec((EDGE_BLK, w), lambda i: (i, 0))
    wb = lambda s: pl.BlockSpec(s, lambda i: (0, 0))
    bf = jnp.bfloat16
    jj = jnp.arange(NS * SH) % SH
    pmat = (jnp.arange(16)[:, None] == jj[None, :]).astype(bf)
    ii = jnp.arange(NS * SH) // SH
    smat = (ii[:, None] == jnp.arange(NS)[None, :]).astype(bf)
    return pl.pallas_call(
        functools.partial(_conv_body, hoff),
        grid=grid,
        in_specs=[eb(NS), eb(LW), eb(LW), eb(16),
                  wb((3 * NS, 3 * NS)), wb((1, 3 * NS)),
                  wb((3 * NS, NS * SH)), wb((1, NS * SH)),
                  wb((16, NS * SH)), wb((NS * SH, NS))],
        out_specs=eb(LW),
        out_shape=jax.ShapeDtypeStruct((e_count, LW), jnp.float32),
        compiler_params=pltpu.CompilerParams(
            dimension_semantics=("parallel",)),
    )(e, hsrc, hdst, sh16, Wc1.T.astype(bf), bc1.reshape(1, -1),
      Wc2.T.astype(bf), bc2.reshape(1, -1), pmat, smat)


def _update_body(final, h_ref, acc_ref, *o_refs):
    nn = h_ref.shape[0]
    agg = acc_ref[0, 0:nn, 0:NS] + acc_ref[1, 0:nn, 0:NS]
    deg = jnp.maximum(
        acc_ref[0, 0:nn, NS:NS + 1] + acc_ref[1, 0:nn, NS:NS + 1], 1.0)
    hn = h_ref[...] + agg / deg
    o_refs[0][...] = hn
    if not final:
        o_refs[1][...] = jnp.concatenate(
            [hn, jnp.zeros((nn, LW - NS), jnp.float32)], axis=1)


def _h_update(h, acc, final):
    n = h.shape[0]
    shapes = [jax.ShapeDtypeStruct((n, NS), jnp.float32)]
    if not final:
        shapes.append(jax.ShapeDtypeStruct((n, LW), jnp.float32))
    out = pl.pallas_call(
        functools.partial(_update_body, final),
        out_shape=shapes,
    )(h, acc)
    return out if not final else (out[0], None)


_SC_CORES = 2
_SC_SUBCORES = 16
_SC_WORKERS = _SC_CORES * _SC_SUBCORES
_STREAM = 80
_CHUNKS = 5


def _sc_mesh():
    return plsc.VectorSubcoreMesh(core_axis_name="c", subcore_axis_name="s")


_N_PAD = 10240


@jax.jit
def _sc_gather2(tab_a, idx_a3, tab_b, idx_b3):
    _, rows_w, _ = idx_a3.shape
    e_count = _SC_WORKERS * rows_w * _STREAM
    dt_a, dt_b = tab_a.dtype, tab_b.dtype

    @functools.partial(
        pl.kernel,
        out_type=[jax.ShapeDtypeStruct((e_count, LW), dt_a),
                  jax.ShapeDtypeStruct((e_count, LW), dt_b)],
        mesh=_sc_mesh(),
        scratch_types=[pltpu.VMEM((rows_w, _STREAM), jnp.int32),
                       pltpu.VMEM((_STREAM, LW), dt_a),
                       pltpu.VMEM((_STREAM, LW), dt_a),
                       pltpu.VMEM((rows_w, _STREAM), jnp.int32),
                       pltpu.VMEM((_STREAM, LW), dt_b),
                       pltpu.VMEM((_STREAM, LW), dt_b),
                       pltpu.SemaphoreType.DMA,
                       pltpu.SemaphoreType.DMA,
                       pltpu.SemaphoreType.DMA,
                       pltpu.SemaphoreType.DMA],
    )
    def k(ta_hbm, ia_hbm, tb_hbm, ib_hbm, oa_hbm, ob_hbm,
          ia_v, ra0_v, ra1_v, ib_v, rb0_v, rb1_v, sa0, sa1, sb0, sb1):
        wid = lax.axis_index("s") * _SC_CORES + lax.axis_index("c")
        base = wid * rows_w * _STREAM
        pltpu.sync_copy(ia_hbm.at[wid], ia_v)
        pltpu.sync_copy(ib_hbm.at[wid], ib_v)

        def row(j, ra, rb, sa, sb):
            off = base + j * _STREAM
            ca = pltpu.async_copy(ta_hbm.at[ia_v.at[j]], ra, sa)
            cb = pltpu.async_copy(tb_hbm.at[ib_v.at[j]], rb, sb)
            return off, ca, cb

        @pl.loop(0, rows_w // 2)
        def _(p):
            j = p * 2
            o0, ca0, cb0 = row(j, ra0_v, rb0_v, sa0, sb0)
            o1, ca1, cb1 = row(j + 1, ra1_v, rb1_v, sa1, sb1)
            ca0.wait()
            pltpu.sync_copy(ra0_v, oa_hbm.at[pl.ds(o0, _STREAM)])
            cb0.wait()
            pltpu.sync_copy(rb0_v, ob_hbm.at[pl.ds(o0, _STREAM)])
            ca1.wait()
            pltpu.sync_copy(ra1_v, oa_hbm.at[pl.ds(o1, _STREAM)])
            cb1.wait()
            pltpu.sync_copy(rb1_v, ob_hbm.at[pl.ds(o1, _STREAM)])

        if rows_w % 2:
            o0, ca0, cb0 = row(rows_w - 1, ra0_v, rb0_v, sa0, sb0)
            ca0.wait()
            pltpu.sync_copy(ra0_v, oa_hbm.at[pl.ds(o0, _STREAM)])
            cb0.wait()
            pltpu.sync_copy(rb0_v, ob_hbm.at[pl.ds(o0, _STREAM)])

    return k(tab_a, idx_a3, tab_b, idx_b3)


@jax.jit
def _sc_scatter(msg, dst3, init):
    _, rows_w, _ = dst3.shape
    stripe = _N_PAD // _SC_SUBCORES

    @functools.partial(
        pl.kernel,
        out_type=jax.ShapeDtypeStruct((_SC_CORES, _N_PAD, LW), jnp.float32),
        mesh=_sc_mesh(),
        scratch_types=[pltpu.VMEM((rows_w, _STREAM), jnp.int32),
                       pltpu.VMEM((_STREAM, LW), jnp.float32),
                       pltpu.VMEM((_STREAM, LW), jnp.float32),
                       pltpu.VMEM_SHARED((_N_PAD, LW), jnp.float32),
                       pltpu.SemaphoreType.DMA,
                       pltpu.SemaphoreType.DMA],
    )
    def k(msg_hbm, dst_hbm, z_hbm, acc_hbm, idx_v, m0_v, m1_v, acc_sh,
          s0, s1):
        c = lax.axis_index("c")
        s = lax.axis_index("s")
        wid = s * _SC_CORES + c
        base = wid * rows_w * _STREAM
        pltpu.sync_copy(z_hbm.at[c, pl.ds(s * stripe, stripe)],
                        acc_sh.at[pl.ds(s * stripe, stripe)])
        pltpu.sync_copy(dst_hbm.at[wid], idx_v)
        plsc.subcore_barrier()

        @pl.loop(0, rows_w // 2)
        def _(p):
            j = p * 2
            c0 = pltpu.async_copy(
                msg_hbm.at[pl.ds(base + j * _STREAM, _STREAM)], m0_v, s0)
            c1 = pltpu.async_copy(
                msg_hbm.at[pl.ds(base + (j + 1) * _STREAM, _STREAM)], m1_v,
                s1)
            c0.wait()
            pltpu.sync_copy(m0_v, acc_sh.at[idx_v.at[j]], add=True)
            c1.wait()
            pltpu.sync_copy(m1_v, acc_sh.at[idx_v.at[j + 1]], add=True)

        if rows_w % 2:
            j = rows_w - 1
            pltpu.sync_copy(msg_hbm.at[pl.ds(base + j * _STREAM, _STREAM)],
                            m0_v)
            pltpu.sync_copy(m0_v, acc_sh.at[idx_v.at[j]], add=True)

        plsc.subcore_barrier()
        pltpu.sync_copy(acc_sh.at[pl.ds(s * stripe, stripe)],
                        acc_hbm.at[c, pl.ds(s * stripe, stripe)])

    return k(msg, dst3, init)


def kernel(x, pos, node_sigma_emb, edge_attr, W_node, b_node, We1, be1, We2,
           be2, Wc1_0, bc1_0, Wc2_0, bc2_0, Wc1_1, bc1_1, Wc2_1, bc2_1,
           edge_index):
    n = x.shape[0]
    e_count = edge_index.shape[1]
    e_chunk = e_count // _CHUNKS
    src3 = [edge_index[0, k * e_chunk:(k + 1) * e_chunk]
            .reshape(_SC_WORKERS, -1, _STREAM) for k in range(_CHUNKS)]
    dst3 = [edge_index[1, k * e_chunk:(k + 1) * e_chunk]
            .reshape(_SC_WORKERS, -1, _STREAM) for k in range(_CHUNKS)]

    h, hb = _node_embed(x, node_sigma_emb, pos, W_node, b_node)
    ea8 = jnp.concatenate(
        [edge_attr, jnp.zeros((e_count, 3), jnp.float32)], axis=1)

    zeros2 = jnp.zeros((_SC_CORES, _N_PAD, LW), jnp.float32)

    e, sh16 = [], []
    g = [_sc_gather2(hb, src3[k], hb, dst3[k]) for k in range(_CHUNKS)]
    acc = zeros2
    for k in range(_CHUNKS):
        tsrc, tdst = g[k]
        ek, shk = _edge_prep(ea8[k * e_chunk:(k + 1) * e_chunk], tsrc, tdst,
                             We1, be1, We2, be2)
        e.append(ek)
        sh16.append(shk)
        msg = _conv_layer(ek, tsrc, tdst, shk, Wc1_0, bc1_0, Wc2_0, bc2_0,
                          hoff=_HOFF)
        acc = _sc_scatter(msg, dst3[k], acc)
    h, hb = _h_update(h, acc, final=False)

    g = [_sc_gather2(hb, src3[k], hb, dst3[k]) for k in range(_CHUNKS)]
    acc = zeros2
    for k in range(_CHUNKS):
        hsrc, hdst = g[k]
        msg = _conv_layer(e[k], hsrc, hdst, sh16[k], Wc1_1, bc1_1,
                          Wc2_1, bc2_1, hoff=0)
        acc = _sc_scatter(msg, dst3[k], acc)
    h, _ = _h_update(h, acc, final=True)
    return h
